# baseline (speedup 1.0000x reference)
import functools

import jax
import jax.numpy as jnp
from jax import lax
from jax.experimental import pallas as pl
from jax.experimental.pallas import tpu as pltpu
from jax.experimental.pallas import tpu_sc as plsc

N = 10000
E = 320000
D = 128

NC = 2
NS = 16
NW = NC * NS
CHUNK = 128
NCHUNK = 80
EPW = CHUNK * NCHUNK
EP = NW * EPW
NPAD = 10240
RPS = NPAD // NS


def _sc_aggregate(feat, src_flat, dst3, zeros):
    mesh = plsc.VectorSubcoreMesh(core_axis_name="c", subcore_axis_name="s")

    @functools.partial(
        pl.kernel,
        out_type=jax.ShapeDtypeStruct((NC * NPAD, D), jnp.float32),
        mesh=mesh,
        scratch_types=[
            pltpu.VMEM((CHUNK,), jnp.int32),
            pltpu.VMEM((CHUNK,), jnp.int32),
            pltpu.VMEM((NCHUNK, CHUNK), jnp.int32),
            pltpu.VMEM((CHUNK, D), jnp.float32),
            pltpu.VMEM((CHUNK, D), jnp.float32),
            pltpu.VMEM_SHARED((NPAD, D), jnp.float32),
            pltpu.SemaphoreType.DMA,
            pltpu.SemaphoreType.DMA,
            pltpu.SemaphoreType.DMA,
            pltpu.SemaphoreType.DMA,
        ],
    )
    def agg_kernel(feat_hbm, src_hbm, dst_hbm, zeros_hbm, out_hbm, sidx_a,
                   sidx_b, didx, rows_a, rows_b, acc, sem_a, sem_b, sem_ia,
                   sem_ib):
        cid = lax.axis_index("c")
        sid = lax.axis_index("s")
        wid = sid * NC + cid
        base = wid * EPW

        pltpu.sync_copy(dst_hbm.at[wid], didx)
        pltpu.sync_copy(zeros_hbm, acc.at[pl.ds(sid * RPS, RPS)])
        pltpu.sync_copy(src_hbm.at[pl.ds(base, CHUNK)], sidx_a)
        pltpu.sync_copy(src_hbm.at[pl.ds(base + CHUNK, CHUNK)], sidx_b)
        pltpu.async_copy(feat_hbm.at[sidx_a], rows_a, sem_a)
        pltpu.async_copy(feat_hbm.at[sidx_b], rows_b, sem_b)
        plsc.subcore_barrier()

        def phase(j, sidx, rows, sem_g, sem_i):
            pltpu.make_async_copy(feat_hbm.at[sidx], rows, sem_g).wait()

            @pl.when(j + 2 < NCHUNK)
            def _():
                pltpu.async_copy(
                    src_hbm.at[pl.ds(base + (j + 2) * CHUNK, CHUNK)],
                    sidx, sem_i)

            pltpu.sync_copy(rows, acc.at[didx.at[j]], add=True)

            @pl.when(j + 2 < NCHUNK)
            def _():
                pltpu.make_async_copy(
                    src_hbm.at[pl.ds(base + (j + 2) * CHUNK, CHUNK)],
                    sidx, sem_i).wait()
                pltpu.async_copy(feat_hbm.at[sidx], rows, sem_g)

        @pl.loop(0, NCHUNK, step=2)
        def _(j):
            phase(j, sidx_a, rows_a, sem_a, sem_ia)
            phase(j + 1, sidx_b, rows_b, sem_b, sem_ib)

        plsc.subcore_barrier()
        pltpu.sync_copy(
            acc.at[pl.ds(sid * RPS, RPS)],
            out_hbm.at[pl.ds(cid * NPAD + sid * RPS, RPS)],
        )

    return agg_kernel(feat, src_flat, dst3, zeros)


def _dense_layer(x, p0, p1, indeg, w_top, w_bot, b, gamma, beta, ln_relu):
    R = 2000

    def body(x_ref, p0_ref, p1_ref, d_ref, wt_ref, wb_ref, b_ref, g_ref,
             be_ref, o_ref):
        agg = (p0_ref[...] + p1_ref[...]) / d_ref[...]
        h = (
            jnp.dot(x_ref[...], wt_ref[...], preferred_element_type=jnp.float32)
            + jnp.dot(agg, wb_ref[...], preferred_element_type=jnp.float32)
            + b_ref[...]
        )
        if ln_relu:
            mu = jnp.mean(h, axis=-1, keepdims=True)
            var = jnp.mean((h - mu) ** 2, axis=-1, keepdims=True)
            h = (h - mu) * lax.rsqrt(var + 1e-5) * g_ref[...] + be_ref[...]
            h = jnp.maximum(h, 0.0)
        o_ref[...] = h

    row_spec = pl.BlockSpec((R, D), lambda i: (i, 0))
    full_spec = pl.BlockSpec((D, D), lambda i: (0, 0))
    vec_spec = pl.BlockSpec((1, D), lambda i: (0, 0))
    return pl.pallas_call(
        body,
        grid=(N // R,),
        in_specs=[
            row_spec, row_spec, row_spec,
            pl.BlockSpec((R, 1), lambda i: (i, 0)),
            full_spec, full_spec, vec_spec, vec_spec, vec_spec,
        ],
        out_specs=row_spec,
        out_shape=jax.ShapeDtypeStruct((N, D), jnp.float32),
    )(x, p0, p1, indeg, w_top, w_bot, b, gamma, beta)


def kernel(feat, g, in_deg, W1, b1, W2, b2, gamma, beta):
    zeros = jnp.zeros((RPS, D), jnp.float32)
    indeg = in_deg[:, None]
    b1r = b1[None, :]
    b2r = b2[None, :]
    gr = gamma[None, :]
    ber = beta[None, :]

    pad = EP - E
    src_flat = jnp.concatenate([g[0], jnp.zeros((pad,), jnp.int32)])
    dst3 = jnp.concatenate([g[1], jnp.full((pad,), N, jnp.int32)]).reshape(
        NW, NCHUNK, CHUNK)
    p = _sc_aggregate(feat, src_flat, dst3, zeros)
    h1 = _dense_layer(feat, p[:N], p[NPAD:NPAD + N], indeg, W1[:D], W1[D:],
                      b1r, gr, ber, True)
    p2 = _sc_aggregate(h1, src_flat, dst3, zeros)
    return _dense_layer(h1, p2[:N], p2[NPAD:NPAD + N], indeg, W2[:D], W2[D:],
                        b2r, gr, ber, False)
